# Initial kernel scaffold; baseline (speedup 1.0000x reference)
import functools

import jax
import jax.numpy as jnp
from jax import lax
from jax.experimental import pallas as pl
from jax.experimental.pallas import tpu as pltpu
from jax.experimental.pallas import tpu_sc as plsc

N = 10000
D = 128
E = 320000
L = 3
C = 10
NB3 = 3

NC = 2
NS = 16
NW = NC * NS

NP = 10240
CH = 128
EPW = 10240
NCH = EPW // CH
EP = NW * EPW
RPT = NP // NS
ZCOPIES = RPT // CH

_MESH = plsc.VectorSubcoreMesh(
    core_axis_name="c", subcore_axis_name="s", num_cores=NC, num_subcores=NS)


@functools.partial(
    pl.kernel,
    out_type=jax.ShapeDtypeStruct((NC, NB3, NP), jnp.float32),
    mesh=_MESH,
    scratch_types=[
        pltpu.VMEM((CH,), jnp.int32),
        pltpu.VMEM((CH,), jnp.float32),
        pltpu.VMEM((RPT,), jnp.float32),
        pltpu.VMEM_SHARED((NP,), jnp.float32),
    ],
)
def _deg_kernel(dst_hbm, degp_hbm, idx_v, ones_v, zed_v, deg_sh):
    c = lax.axis_index("c")
    s = lax.axis_index("s")
    wid = s * NC + c

    for j in range(CH // 16):
        ones_v[pl.ds(j * 16, 16)] = jnp.ones((16,), jnp.float32)

    def _zfill(i, carry):
        zed_v[pl.ds(i * 16, 16)] = jnp.zeros((16,), jnp.float32)
        return carry

    lax.fori_loop(0, RPT // 16, _zfill, 0)

    for b in range(NB3):
        pltpu.sync_copy(zed_v, deg_sh.at[pl.ds(s * RPT, RPT)])
        plsc.subcore_barrier()

        def _chunk(k, carry):
            pltpu.sync_copy(dst_hbm.at[b, wid, k], idx_v)
            pltpu.sync_copy(ones_v, deg_sh.at[idx_v], add=True)
            return carry

        lax.fori_loop(0, NCH, _chunk, 0)
        plsc.subcore_barrier()
        pltpu.sync_copy(deg_sh.at[pl.ds(s * RPT, RPT)],
                        degp_hbm.at[c, b, pl.ds(s * RPT, RPT)])
        plsc.subcore_barrier()


@functools.partial(
    pl.kernel,
    out_type=jax.ShapeDtypeStruct((NC, NB3, NP, D), jnp.float32),
    mesh=_MESH,
    scratch_types=[
        pltpu.VMEM((CH,), jnp.int32),
        pltpu.VMEM((CH,), jnp.int32),
        pltpu.VMEM((CH, D), jnp.float32),
        pltpu.VMEM((CH, D), jnp.float32),
        pltpu.VMEM_SHARED((NP, D), jnp.float32),
    ],
)
def _acc_kernel(u_hbm, src_hbm, dst_hbm, accp_hbm,
                si_v, di_v, rows_v, zbuf_v, acc_sh):
    c = lax.axis_index("c")
    s = lax.axis_index("s")
    wid = s * NC + c

    def _zfill(i, carry):
        for j in range(D // 16):
            zbuf_v[i, pl.ds(j * 16, 16)] = jnp.zeros((16,), jnp.float32)
        return carry

    lax.fori_loop(0, CH, _zfill, 0)

    for b in range(NB3):
        for r in range(ZCOPIES):
            pltpu.sync_copy(zbuf_v, acc_sh.at[pl.ds(s * RPT + r * CH, CH), :])
        plsc.subcore_barrier()

        def _chunk(k, carry):
            pltpu.sync_copy(src_hbm.at[b, wid, k], si_v)
            pltpu.sync_copy(dst_hbm.at[b, wid, k], di_v)
            pltpu.sync_copy(u_hbm.at[si_v], rows_v)
            pltpu.sync_copy(rows_v, acc_sh.at[di_v], add=True)
            return carry

        lax.fori_loop(0, NCH, _chunk, 0)
        plsc.subcore_barrier()
        pltpu.sync_copy(acc_sh.at[pl.ds(s * RPT, RPT), :],
                        accp_hbm.at[c, b, pl.ds(s * RPT, RPT), :])
        plsc.subcore_barrier()


def _sc_degrees(dst_f):
    return _deg_kernel(dst_f)


def _sc_accumulate(u2, src_f, dst_f):
    return _acc_kernel(u2, src_f, dst_f)


RB = 1280
NBLK = NP // RB
EPS = 1e-5


def _init_body(x_ref, w_ref, degp_ref, u_ref, dinv_ref):
    deg = degp_ref[0, 0] + degp_ref[1, 0] + 1.0
    dinv = lax.rsqrt(deg)
    h = jnp.dot(x_ref[0], w_ref[0], preferred_element_type=jnp.float32)
    u_ref[0] = h * dinv
    dinv_ref[0] = dinv


def _layer_update(x, u, acc0, acc1, dinv, bb, gg, be):
    a = (acc0 + acc1 + u) * dinv + bb
    mu = jnp.mean(a, axis=-1, keepdims=True)
    var = jnp.mean((a - mu) ** 2, axis=-1, keepdims=True)
    hn = (a - mu) * lax.rsqrt(var + EPS) * gg + be
    return x + jnp.maximum(hn, 0.0)


def _mid_body(x_ref, u_ref, accp_ref, dinv_ref, b_ref, g_ref, be_ref, wn_ref,
              xn_ref, un_ref):
    dinv = dinv_ref[0]
    xn = _layer_update(x_ref[0], u_ref[0], accp_ref[0, 0], accp_ref[1, 0],
                       dinv, b_ref[0], g_ref[0], be_ref[0])
    xn_ref[0] = xn
    un_ref[0] = jnp.dot(xn, wn_ref[0],
                        preferred_element_type=jnp.float32) * dinv


def _fin_body(x_ref, u_ref, accp_ref, dinv_ref, b_ref, g_ref, be_ref,
              cw_ref, cb_ref, out_ref):
    acc = jnp.zeros((RB, C), jnp.float32)
    for b3 in range(NB3):
        xn = _layer_update(x_ref[b3], u_ref[b3],
                           accp_ref[0, b3], accp_ref[1, b3],
                           dinv_ref[b3], b_ref[b3], g_ref[b3], be_ref[b3])
        acc = acc + jnp.dot(xn, cw_ref[b3],
                            preferred_element_type=jnp.float32)
    out_ref[...] = acc + cb_ref[...]


_f_init = pl.pallas_call(
    _init_body,
    grid=(NB3, NBLK),
    in_specs=[
        pl.BlockSpec((1, RB, D), lambda b, i: (b, i, 0)),
        pl.BlockSpec((1, D, D), lambda b, i: (b, 0, 0)),
        pl.BlockSpec((NC, 1, RB, 1), lambda b, i: (0, b, i, 0)),
    ],
    out_specs=[
        pl.BlockSpec((1, RB, D), lambda b, i: (b, i, 0)),
        pl.BlockSpec((1, RB, 1), lambda b, i: (b, i, 0)),
    ],
    out_shape=[
        jax.ShapeDtypeStruct((NB3, NP, D), jnp.float32),
        jax.ShapeDtypeStruct((NB3, NP, 1), jnp.float32),
    ],
)

_f_mid = pl.pallas_call(
    _mid_body,
    grid=(NB3, NBLK),
    in_specs=[
        pl.BlockSpec((1, RB, D), lambda b, i: (b, i, 0)),
        pl.BlockSpec((1, RB, D), lambda b, i: (b, i, 0)),
        pl.BlockSpec((NC, 1, RB, D), lambda b, i: (0, b, i, 0)),
        pl.BlockSpec((1, RB, 1), lambda b, i: (b, i, 0)),
        pl.BlockSpec((1, 1, D), lambda b, i: (b, 0, 0)),
        pl.BlockSpec((1, 1, D), lambda b, i: (b, 0, 0)),
        pl.BlockSpec((1, 1, D), lambda b, i: (b, 0, 0)),
        pl.BlockSpec((1, D, D), lambda b, i: (b, 0, 0)),
    ],
    out_specs=[
        pl.BlockSpec((1, RB, D), lambda b, i: (b, i, 0)),
        pl.BlockSpec((1, RB, D), lambda b, i: (b, i, 0)),
    ],
    out_shape=[
        jax.ShapeDtypeStruct((NB3, NP, D), jnp.float32),
        jax.ShapeDtypeStruct((NB3, NP, D), jnp.float32),
    ],
)

_f_fin = pl.pallas_call(
    _fin_body,
    grid=(NBLK,),
    in_specs=[
        pl.BlockSpec((NB3, RB, D), lambda i: (0, i, 0)),
        pl.BlockSpec((NB3, RB, D), lambda i: (0, i, 0)),
        pl.BlockSpec((NC, NB3, RB, D), lambda i: (0, 0, i, 0)),
        pl.BlockSpec((NB3, RB, 1), lambda i: (0, i, 0)),
        pl.BlockSpec((NB3, 1, D), lambda i: (0, 0, 0)),
        pl.BlockSpec((NB3, 1, D), lambda i: (0, 0, 0)),
        pl.BlockSpec((NB3, 1, D), lambda i: (0, 0, 0)),
        pl.BlockSpec((NB3, D, C), lambda i: (0, 0, 0)),
        pl.BlockSpec((1, C), lambda i: (0, 0)),
    ],
    out_specs=pl.BlockSpec((RB, C), lambda i: (i, 0)),
    out_shape=jax.ShapeDtypeStruct((NP, C), jnp.float32),
)


def kernel(x_renormalized, edge_index_renormalized, x_vanilla,
           edge_index_vanilla, x_third, edge_index_third,
           W_ren, b_ren, g_ren, be_ren, W_van, b_van, g_van, be_van,
           W_thd, b_thd, g_thd, be_thd, clf_W, clf_b):
    xs = jnp.stack([x_renormalized, x_vanilla, x_third])
    xs = jnp.pad(xs, ((0, 0), (0, NP - N), (0, 0)))
    eis = jnp.stack([edge_index_renormalized, edge_index_vanilla,
                     edge_index_third])
    src = eis[:, 0, :]
    dst = eis[:, 1, :]

    pad_n = EP - E
    pad_idx = (jnp.arange(pad_n, dtype=jnp.int32) % 16) + N
    pad_blk = jnp.broadcast_to(pad_idx, (NB3, pad_n))
    src_p = jnp.concatenate([src, pad_blk], axis=1)
    dst_p = jnp.concatenate([dst, pad_blk], axis=1)
    boff = (jnp.arange(NB3, dtype=jnp.int32) * NP)[:, None]
    src_f = (src_p + boff).reshape(NB3, NW, NCH, CH)
    dst_f = dst_p.reshape(NB3, NW, NCH, CH)

    W_all = jnp.stack([W_ren, W_van, W_thd])
    b_all = jnp.stack([b_ren, b_van, b_thd])
    g_all = jnp.stack([g_ren, g_van, g_thd])
    be_all = jnp.stack([be_ren, be_van, be_thd])
    cw = clf_W.reshape(NB3, D, C)
    cb = clf_b.reshape(1, C)

    degp = _sc_degrees(dst_f)
    u, dinv = _f_init(xs, W_all[:, 0], degp.reshape(NC, NB3, NP, 1))

    x_cur = xs
    out = None
    for l in range(L):
        accp = _sc_accumulate(u.reshape(NB3 * NP, D), src_f, dst_f)
        bl = b_all[:, l].reshape(NB3, 1, D)
        gl = g_all[:, l].reshape(NB3, 1, D)
        bel = be_all[:, l].reshape(NB3, 1, D)
        if l < L - 1:
            x_cur, u = _f_mid(x_cur, u, accp, dinv, bl, gl, bel,
                              W_all[:, l + 1])
        else:
            out = _f_fin(x_cur, u, accp, dinv, bl, gl, bel, cw, cb)
    return out[:N]


# speedup vs baseline: 10.3519x; 10.3519x over previous
import functools

import jax
import jax.numpy as jnp
from jax import lax
from jax.experimental import pallas as pl
from jax.experimental.pallas import tpu as pltpu
from jax.experimental.pallas import tpu_sc as plsc

N = 10000
D = 128
E = 320000
L = 3
C = 10
NB3 = 3

NC = 2
NS = 16
NW = NC * NS

NP = 10240
CH = 128
EPW = 10240
NCH = EPW // CH
EP = NW * EPW
RPT = NP // NS
ZCOPIES = RPT // CH


def _deg_body(dst_hbm, degp_hbm, idx_v, ones_v, zed_v, deg_sh):
    c = lax.axis_index("c")
    s = lax.axis_index("s")
    wid = s * NC + c

    for j in range(CH // 16):
        ones_v[pl.ds(j * 16, 16)] = jnp.ones((16,), jnp.float32)

    def _zfill(i, carry):
        zed_v[pl.ds(i * 16, 16)] = jnp.zeros((16,), jnp.float32)
        return carry

    lax.fori_loop(0, RPT // 16, _zfill, 0)

    for b in range(NB3):
        pltpu.sync_copy(zed_v, deg_sh.at[pl.ds(s * RPT, RPT)])
        plsc.subcore_barrier()
        ebase = (b * NW + wid) * EPW

        def _chunk(k, carry):
            pltpu.sync_copy(dst_hbm.at[pl.ds(ebase + k * CH, CH)], idx_v)
            pltpu.sync_copy(ones_v, deg_sh.at[idx_v], add=True)
            return carry

        lax.fori_loop(0, NCH, _chunk, 0)
        plsc.subcore_barrier()
        pltpu.sync_copy(deg_sh.at[pl.ds(s * RPT, RPT)],
                        degp_hbm.at[pl.ds((c * NB3 + b) * NP + s * RPT, RPT)])
        plsc.subcore_barrier()


def _acc_body(u_hbm, src_hbm, dst_hbm, accp_hbm,
              si_v, di_v, rows_v, zbuf_v, acc_sh):
    c = lax.axis_index("c")
    s = lax.axis_index("s")
    wid = s * NC + c

    def _zfill(i, carry):
        for j in range(D // 16):
            zbuf_v[i, pl.ds(j * 16, 16)] = jnp.zeros((16,), jnp.float32)
        return carry

    lax.fori_loop(0, CH, _zfill, 0)

    for b in range(NB3):
        for r in range(ZCOPIES):
            pltpu.sync_copy(zbuf_v, acc_sh.at[pl.ds(s * RPT + r * CH, CH), :])
        plsc.subcore_barrier()
        ebase = (b * NW + wid) * EPW

        def _chunk(k, carry):
            pltpu.sync_copy(src_hbm.at[pl.ds(ebase + k * CH, CH)], si_v)
            pltpu.sync_copy(dst_hbm.at[pl.ds(ebase + k * CH, CH)], di_v)
            pltpu.sync_copy(u_hbm.at[si_v], rows_v)
            pltpu.sync_copy(rows_v, acc_sh.at[di_v], add=True)
            return carry

        lax.fori_loop(0, NCH, _chunk, 0)
        plsc.subcore_barrier()
        rbase = (c * NB3 + b) * NP + s * RPT
        pltpu.sync_copy(acc_sh.at[pl.ds(s * RPT, RPT), :],
                        accp_hbm.at[pl.ds(rbase, RPT), :])
        plsc.subcore_barrier()


@functools.cache
def _build_sc_kernels():
    mesh = plsc.VectorSubcoreMesh(
        core_axis_name="c", subcore_axis_name="s",
        num_cores=NC, num_subcores=NS)
    deg_k = pl.kernel(
        _deg_body,
        out_type=jax.ShapeDtypeStruct((NC * NB3 * NP,), jnp.float32),
        mesh=mesh,
        scratch_types=[
            pltpu.VMEM((CH,), jnp.int32),
            pltpu.VMEM((CH,), jnp.float32),
            pltpu.VMEM((RPT,), jnp.float32),
            pltpu.VMEM_SHARED((NP,), jnp.float32),
        ],
    )
    acc_k = pl.kernel(
        _acc_body,
        out_type=jax.ShapeDtypeStruct((NC * NB3 * NP, D), jnp.float32),
        mesh=mesh,
        scratch_types=[
            pltpu.VMEM((CH,), jnp.int32),
            pltpu.VMEM((CH,), jnp.int32),
            pltpu.VMEM((CH, D), jnp.float32),
            pltpu.VMEM((CH, D), jnp.float32),
            pltpu.VMEM_SHARED((NP, D), jnp.float32),
        ],
    )
    return deg_k, acc_k


def _sc_degrees(dst_f):
    degp = _build_sc_kernels()[0](dst_f.reshape(-1))
    return degp.reshape(NC, NB3, NP)


def _sc_accumulate(u2, src_f, dst_f):
    accp = _build_sc_kernels()[1](u2, src_f.reshape(-1), dst_f.reshape(-1))
    return accp.reshape(NC, NB3, NP, D)


RB = 1280
NBLK = NP // RB
EPS = 1e-5


def _init_body(x_ref, w_ref, degp_ref, u_ref, dinv_ref):
    deg = degp_ref[0, 0] + degp_ref[1, 0] + 1.0
    dinv = lax.rsqrt(deg)
    h = jnp.dot(x_ref[0], w_ref[0], preferred_element_type=jnp.float32)
    u_ref[0] = h * dinv
    dinv_ref[0] = dinv


def _layer_update(x, u, acc0, acc1, dinv, bb, gg, be):
    a = (acc0 + acc1 + u) * dinv + bb
    mu = jnp.mean(a, axis=-1, keepdims=True)
    var = jnp.mean((a - mu) ** 2, axis=-1, keepdims=True)
    hn = (a - mu) * lax.rsqrt(var + EPS) * gg + be
    return x + jnp.maximum(hn, 0.0)


def _mid_body(x_ref, u_ref, accp_ref, dinv_ref, b_ref, g_ref, be_ref, wn_ref,
              xn_ref, un_ref):
    dinv = dinv_ref[0]
    xn = _layer_update(x_ref[0], u_ref[0], accp_ref[0, 0], accp_ref[1, 0],
                       dinv, b_ref[0], g_ref[0], be_ref[0])
    xn_ref[0] = xn
    un_ref[0] = jnp.dot(xn, wn_ref[0],
                        preferred_element_type=jnp.float32) * dinv


def _fin_body(x_ref, u_ref, accp_ref, dinv_ref, b_ref, g_ref, be_ref,
              cw_ref, cb_ref, out_ref):
    acc = jnp.zeros((RB, C), jnp.float32)
    for b3 in range(NB3):
        xn = _layer_update(x_ref[b3], u_ref[b3],
                           accp_ref[0, b3], accp_ref[1, b3],
                           dinv_ref[b3], b_ref[b3], g_ref[b3], be_ref[b3])
        acc = acc + jnp.dot(xn, cw_ref[b3],
                            preferred_element_type=jnp.float32)
    out_ref[...] = acc + cb_ref[...]


_f_init = pl.pallas_call(
    _init_body,
    grid=(NB3, NBLK),
    in_specs=[
        pl.BlockSpec((1, RB, D), lambda b, i: (b, i, 0)),
        pl.BlockSpec((1, D, D), lambda b, i: (b, 0, 0)),
        pl.BlockSpec((NC, 1, RB, 1), lambda b, i: (0, b, i, 0)),
    ],
    out_specs=[
        pl.BlockSpec((1, RB, D), lambda b, i: (b, i, 0)),
        pl.BlockSpec((1, RB, 1), lambda b, i: (b, i, 0)),
    ],
    out_shape=[
        jax.ShapeDtypeStruct((NB3, NP, D), jnp.float32),
        jax.ShapeDtypeStruct((NB3, NP, 1), jnp.float32),
    ],
)

_f_mid = pl.pallas_call(
    _mid_body,
    grid=(NB3, NBLK),
    in_specs=[
        pl.BlockSpec((1, RB, D), lambda b, i: (b, i, 0)),
        pl.BlockSpec((1, RB, D), lambda b, i: (b, i, 0)),
        pl.BlockSpec((NC, 1, RB, D), lambda b, i: (0, b, i, 0)),
        pl.BlockSpec((1, RB, 1), lambda b, i: (b, i, 0)),
        pl.BlockSpec((1, 1, D), lambda b, i: (b, 0, 0)),
        pl.BlockSpec((1, 1, D), lambda b, i: (b, 0, 0)),
        pl.BlockSpec((1, 1, D), lambda b, i: (b, 0, 0)),
        pl.BlockSpec((1, D, D), lambda b, i: (b, 0, 0)),
    ],
    out_specs=[
        pl.BlockSpec((1, RB, D), lambda b, i: (b, i, 0)),
        pl.BlockSpec((1, RB, D), lambda b, i: (b, i, 0)),
    ],
    out_shape=[
        jax.ShapeDtypeStruct((NB3, NP, D), jnp.float32),
        jax.ShapeDtypeStruct((NB3, NP, D), jnp.float32),
    ],
)

_f_fin = pl.pallas_call(
    _fin_body,
    grid=(NBLK,),
    in_specs=[
        pl.BlockSpec((NB3, RB, D), lambda i: (0, i, 0)),
        pl.BlockSpec((NB3, RB, D), lambda i: (0, i, 0)),
        pl.BlockSpec((NC, NB3, RB, D), lambda i: (0, 0, i, 0)),
        pl.BlockSpec((NB3, RB, 1), lambda i: (0, i, 0)),
        pl.BlockSpec((NB3, 1, D), lambda i: (0, 0, 0)),
        pl.BlockSpec((NB3, 1, D), lambda i: (0, 0, 0)),
        pl.BlockSpec((NB3, 1, D), lambda i: (0, 0, 0)),
        pl.BlockSpec((NB3, D, C), lambda i: (0, 0, 0)),
        pl.BlockSpec((1, C), lambda i: (0, 0)),
    ],
    out_specs=pl.BlockSpec((RB, C), lambda i: (i, 0)),
    out_shape=jax.ShapeDtypeStruct((NP, C), jnp.float32),
)


def kernel(x_renormalized, edge_index_renormalized, x_vanilla,
           edge_index_vanilla, x_third, edge_index_third,
           W_ren, b_ren, g_ren, be_ren, W_van, b_van, g_van, be_van,
           W_thd, b_thd, g_thd, be_thd, clf_W, clf_b):
    xs = jnp.stack([x_renormalized, x_vanilla, x_third])
    xs = jnp.pad(xs, ((0, 0), (0, NP - N), (0, 0)))
    eis = jnp.stack([edge_index_renormalized, edge_index_vanilla,
                     edge_index_third])
    src = eis[:, 0, :]
    dst = eis[:, 1, :]

    pad_n = EP - E
    pad_idx = (jnp.arange(pad_n, dtype=jnp.int32) % 16) + N
    pad_blk = jnp.broadcast_to(pad_idx, (NB3, pad_n))
    src_p = jnp.concatenate([src, pad_blk], axis=1)
    dst_p = jnp.concatenate([dst, pad_blk], axis=1)
    boff = (jnp.arange(NB3, dtype=jnp.int32) * NP)[:, None]
    src_f = (src_p + boff).reshape(NB3, NW, NCH, CH)
    dst_f = dst_p.reshape(NB3, NW, NCH, CH)

    W_all = jnp.stack([W_ren, W_van, W_thd])
    b_all = jnp.stack([b_ren, b_van, b_thd])
    g_all = jnp.stack([g_ren, g_van, g_thd])
    be_all = jnp.stack([be_ren, be_van, be_thd])
    cw = clf_W.reshape(NB3, D, C)
    cb = clf_b.reshape(1, C)

    degp = _sc_degrees(dst_f)
    u, dinv = _f_init(xs, W_all[:, 0], degp.reshape(NC, NB3, NP, 1))

    x_cur = xs
    out = None
    for l in range(L):
        accp = _sc_accumulate(u.reshape(NB3 * NP, D), src_f, dst_f)
        bl = b_all[:, l].reshape(NB3, 1, D)
        gl = g_all[:, l].reshape(NB3, 1, D)
        bel = be_all[:, l].reshape(NB3, 1, D)
        if l < L - 1:
            x_cur, u = _f_mid(x_cur, u, accp, dinv, bl, gl, bel,
                              W_all[:, l + 1])
        else:
            out = _f_fin(x_cur, u, accp, dinv, bl, gl, bel, cw, cb)
    return out[:N]


# speedup vs baseline: 21.0252x; 2.0310x over previous
import functools

import jax
import jax.numpy as jnp
from jax import lax
from jax.experimental import pallas as pl
from jax.experimental.pallas import tpu as pltpu
from jax.experimental.pallas import tpu_sc as plsc

N = 10000
D = 128
E = 320000
L = 3
C = 10
NB3 = 3

NC = 2
NS = 16
NW = NC * NS

NP = 10240
CH = 128
EPW = 10240
NCH = EPW // CH
EP = NW * EPW
RPT = NP // NS
ZCOPIES = RPT // CH
NCHH = NCH // 2


def _deg_body(dst_hbm, degp_hbm, idx_all, ones_v, zed_v, deg_sh, sem):
    c = lax.axis_index("c")
    s = lax.axis_index("s")
    wid = s * NC + c

    for j in range(CH // 16):
        ones_v[pl.ds(j * 16, 16)] = jnp.ones((16,), jnp.float32)

    def _zfill(i, carry):
        zed_v[pl.ds(i * 16, 16)] = jnp.zeros((16,), jnp.float32)
        return carry

    lax.fori_loop(0, RPT // 16, _zfill, 0)

    for b in range(NB3):
        pltpu.sync_copy(dst_hbm.at[b * NW + wid], idx_all)
        pltpu.sync_copy(zed_v, deg_sh.at[pl.ds(s * RPT, RPT)])
        plsc.subcore_barrier()

        def _fire(k, carry):
            pltpu.async_copy(ones_v, deg_sh.at[idx_all.at[k]], sem, add=True)
            return carry

        lax.fori_loop(0, NCH, _fire, 0)

        def _drain(k, carry):
            pltpu.make_async_copy(
                ones_v, deg_sh.at[idx_all.at[k]], sem).wait()
            return carry

        lax.fori_loop(0, NCH, _drain, 0)
        plsc.subcore_barrier()
        pltpu.sync_copy(deg_sh.at[pl.ds(s * RPT, RPT)],
                        degp_hbm.at[pl.ds((c * NB3 + b) * NP + s * RPT, RPT)])
        plsc.subcore_barrier()


def _acc_body(u_hbm, src_hbm, dst_hbm, accp_hbm,
              src_half, dst_half, rows_a, rows_b, acc_sh,
              sem_a, sem_b):
    c = lax.axis_index("c")
    s = lax.axis_index("s")
    wid = s * NC + c

    for b in range(NB3):
        def _zfill(i, carry):
            for j in range(D // 16):
                rows_a[i, pl.ds(j * 16, 16)] = jnp.zeros((16,), jnp.float32)
            return carry

        lax.fori_loop(0, CH, _zfill, 0)
        for r in range(ZCOPIES):
            pltpu.sync_copy(rows_a, acc_sh.at[pl.ds(s * RPT + r * CH, CH), :])
        plsc.subcore_barrier()

        for h in range(2):
            pltpu.sync_copy(
                src_hbm.at[b * NW + wid, pl.ds(h * NCHH, NCHH), :], src_half)
            pltpu.sync_copy(
                dst_hbm.at[b * NW + wid, pl.ds(h * NCHH, NCHH), :], dst_half)

            pltpu.async_copy(u_hbm.at[src_half.at[0]], rows_a, sem_a)
            pltpu.async_copy(u_hbm.at[src_half.at[1]], rows_b, sem_b)

            def _pipe(g, carry):
                k = 2 * g
                pltpu.make_async_copy(u_hbm.at[src_half.at[k]],
                                      rows_a, sem_a).wait()
                pltpu.sync_copy(rows_a, acc_sh.at[dst_half.at[k]], add=True)

                @pl.when(k + 2 < NCHH)
                def _():
                    pltpu.async_copy(u_hbm.at[src_half.at[k + 2]],
                                     rows_a, sem_a)

                pltpu.make_async_copy(u_hbm.at[src_half.at[k + 1]],
                                      rows_b, sem_b).wait()
                pltpu.sync_copy(rows_b, acc_sh.at[dst_half.at[k + 1]],
                                add=True)

                @pl.when(k + 3 < NCHH)
                def _():
                    pltpu.async_copy(u_hbm.at[src_half.at[k + 3]],
                                     rows_b, sem_b)

                return carry

            lax.fori_loop(0, NCHH // 2, _pipe, 0)

        plsc.subcore_barrier()
        rbase = (c * NB3 + b) * NP + s * RPT
        pltpu.sync_copy(acc_sh.at[pl.ds(s * RPT, RPT), :],
                        accp_hbm.at[pl.ds(rbase, RPT), :])
        plsc.subcore_barrier()


@functools.cache
def _build_sc_kernels():
    mesh = plsc.VectorSubcoreMesh(
        core_axis_name="c", subcore_axis_name="s",
        num_cores=NC, num_subcores=NS)
    deg_k = pl.kernel(
        _deg_body,
        out_type=jax.ShapeDtypeStruct((NC * NB3 * NP,), jnp.float32),
        mesh=mesh,
        scratch_types=[
            pltpu.VMEM((NCH, CH), jnp.int32),
            pltpu.VMEM((CH,), jnp.float32),
            pltpu.VMEM((RPT,), jnp.float32),
            pltpu.VMEM_SHARED((NP,), jnp.float32),
            pltpu.SemaphoreType.DMA,
        ],
    )
    acc_k = pl.kernel(
        _acc_body,
        out_type=jax.ShapeDtypeStruct((NC * NB3 * NP, D), jnp.float32),
        mesh=mesh,
        scratch_types=[
            pltpu.VMEM((NCHH, CH), jnp.int32),
            pltpu.VMEM((NCHH, CH), jnp.int32),
            pltpu.VMEM((CH, D), jnp.float32),
            pltpu.VMEM((CH, D), jnp.float32),
            pltpu.VMEM_SHARED((NP, D), jnp.float32),
            pltpu.SemaphoreType.DMA,
            pltpu.SemaphoreType.DMA,
        ],
    )
    return deg_k, acc_k


def _sc_degrees(dst_f):
    degp = _build_sc_kernels()[0](dst_f.reshape(NB3 * NW, NCH, CH))
    return degp.reshape(NC, NB3, NP)


def _sc_accumulate(u2, src_f, dst_f):
    accp = _build_sc_kernels()[1](u2, src_f.reshape(NB3 * NW, NCH, CH),
                                  dst_f.reshape(NB3 * NW, NCH, CH))
    return accp.reshape(NC, NB3, NP, D)


RB = 1280
NBLK = NP // RB
EPS = 1e-5


def _init_body(x_ref, w_ref, degp_ref, u_ref, dinv_ref):
    deg = degp_ref[0, 0] + degp_ref[1, 0] + 1.0
    dinv = lax.rsqrt(deg)
    h = jnp.dot(x_ref[0], w_ref[0], preferred_element_type=jnp.float32)
    u_ref[0] = h * dinv
    dinv_ref[0] = dinv


def _layer_update(x, u, acc0, acc1, dinv, bb, gg, be):
    a = (acc0 + acc1 + u) * dinv + bb
    mu = jnp.mean(a, axis=-1, keepdims=True)
    var = jnp.mean((a - mu) ** 2, axis=-1, keepdims=True)
    hn = (a - mu) * lax.rsqrt(var + EPS) * gg + be
    return x + jnp.maximum(hn, 0.0)


def _mid_body(x_ref, u_ref, accp_ref, dinv_ref, b_ref, g_ref, be_ref, wn_ref,
              xn_ref, un_ref):
    dinv = dinv_ref[0]
    xn = _layer_update(x_ref[0], u_ref[0], accp_ref[0, 0], accp_ref[1, 0],
                       dinv, b_ref[0], g_ref[0], be_ref[0])
    xn_ref[0] = xn
    un_ref[0] = jnp.dot(xn, wn_ref[0],
                        preferred_element_type=jnp.float32) * dinv


def _fin_body(x_ref, u_ref, accp_ref, dinv_ref, b_ref, g_ref, be_ref,
              cw_ref, cb_ref, out_ref):
    acc = jnp.zeros((RB, C), jnp.float32)
    for b3 in range(NB3):
        xn = _layer_update(x_ref[b3], u_ref[b3],
                           accp_ref[0, b3], accp_ref[1, b3],
                           dinv_ref[b3], b_ref[b3], g_ref[b3], be_ref[b3])
        acc = acc + jnp.dot(xn, cw_ref[b3],
                            preferred_element_type=jnp.float32)
    out_ref[...] = acc + cb_ref[...]


_f_init = pl.pallas_call(
    _init_body,
    grid=(NB3, NBLK),
    in_specs=[
        pl.BlockSpec((1, RB, D), lambda b, i: (b, i, 0)),
        pl.BlockSpec((1, D, D), lambda b, i: (b, 0, 0)),
        pl.BlockSpec((NC, 1, RB, 1), lambda b, i: (0, b, i, 0)),
    ],
    out_specs=[
        pl.BlockSpec((1, RB, D), lambda b, i: (b, i, 0)),
        pl.BlockSpec((1, RB, 1), lambda b, i: (b, i, 0)),
    ],
    out_shape=[
        jax.ShapeDtypeStruct((NB3, NP, D), jnp.float32),
        jax.ShapeDtypeStruct((NB3, NP, 1), jnp.float32),
    ],
)

_f_mid = pl.pallas_call(
    _mid_body,
    grid=(NB3, NBLK),
    in_specs=[
        pl.BlockSpec((1, RB, D), lambda b, i: (b, i, 0)),
        pl.BlockSpec((1, RB, D), lambda b, i: (b, i, 0)),
        pl.BlockSpec((NC, 1, RB, D), lambda b, i: (0, b, i, 0)),
        pl.BlockSpec((1, RB, 1), lambda b, i: (b, i, 0)),
        pl.BlockSpec((1, 1, D), lambda b, i: (b, 0, 0)),
        pl.BlockSpec((1, 1, D), lambda b, i: (b, 0, 0)),
        pl.BlockSpec((1, 1, D), lambda b, i: (b, 0, 0)),
        pl.BlockSpec((1, D, D), lambda b, i: (b, 0, 0)),
    ],
    out_specs=[
        pl.BlockSpec((1, RB, D), lambda b, i: (b, i, 0)),
        pl.BlockSpec((1, RB, D), lambda b, i: (b, i, 0)),
    ],
    out_shape=[
        jax.ShapeDtypeStruct((NB3, NP, D), jnp.float32),
        jax.ShapeDtypeStruct((NB3, NP, D), jnp.float32),
    ],
)

_f_fin = pl.pallas_call(
    _fin_body,
    grid=(NBLK,),
    in_specs=[
        pl.BlockSpec((NB3, RB, D), lambda i: (0, i, 0)),
        pl.BlockSpec((NB3, RB, D), lambda i: (0, i, 0)),
        pl.BlockSpec((NC, NB3, RB, D), lambda i: (0, 0, i, 0)),
        pl.BlockSpec((NB3, RB, 1), lambda i: (0, i, 0)),
        pl.BlockSpec((NB3, 1, D), lambda i: (0, 0, 0)),
        pl.BlockSpec((NB3, 1, D), lambda i: (0, 0, 0)),
        pl.BlockSpec((NB3, 1, D), lambda i: (0, 0, 0)),
        pl.BlockSpec((NB3, D, C), lambda i: (0, 0, 0)),
        pl.BlockSpec((1, C), lambda i: (0, 0)),
    ],
    out_specs=pl.BlockSpec((RB, C), lambda i: (i, 0)),
    out_shape=jax.ShapeDtypeStruct((NP, C), jnp.float32),
)


def kernel(x_renormalized, edge_index_renormalized, x_vanilla,
           edge_index_vanilla, x_third, edge_index_third,
           W_ren, b_ren, g_ren, be_ren, W_van, b_van, g_van, be_van,
           W_thd, b_thd, g_thd, be_thd, clf_W, clf_b):
    xs = jnp.stack([x_renormalized, x_vanilla, x_third])
    xs = jnp.pad(xs, ((0, 0), (0, NP - N), (0, 0)))
    eis = jnp.stack([edge_index_renormalized, edge_index_vanilla,
                     edge_index_third])
    src = eis[:, 0, :]
    dst = eis[:, 1, :]

    pad_n = EP - E
    pad_idx = (jnp.arange(pad_n, dtype=jnp.int32) % 16) + N
    pad_blk = jnp.broadcast_to(pad_idx, (NB3, pad_n))
    src_p = jnp.concatenate([src, pad_blk], axis=1)
    dst_p = jnp.concatenate([dst, pad_blk], axis=1)
    boff = (jnp.arange(NB3, dtype=jnp.int32) * NP)[:, None]
    src_f = (src_p + boff).reshape(NB3, NW, NCH, CH)
    dst_f = dst_p.reshape(NB3, NW, NCH, CH)

    W_all = jnp.stack([W_ren, W_van, W_thd])
    b_all = jnp.stack([b_ren, b_van, b_thd])
    g_all = jnp.stack([g_ren, g_van, g_thd])
    be_all = jnp.stack([be_ren, be_van, be_thd])
    cw = clf_W.reshape(NB3, D, C)
    cb = clf_b.reshape(1, C)

    degp = _sc_degrees(dst_f)
    u, dinv = _f_init(xs, W_all[:, 0], degp.reshape(NC, NB3, NP, 1))

    x_cur = xs
    out = None
    for l in range(L):
        accp = _sc_accumulate(u.reshape(NB3 * NP, D), src_f, dst_f)
        bl = b_all[:, l].reshape(NB3, 1, D)
        gl = g_all[:, l].reshape(NB3, 1, D)
        bel = be_all[:, l].reshape(NB3, 1, D)
        if l < L - 1:
            x_cur, u = _f_mid(x_cur, u, accp, dinv, bl, gl, bel,
                              W_all[:, l + 1])
        else:
            out = _f_fin(x_cur, u, accp, dinv, bl, gl, bel, cw, cb)
    return out[:N]


# speedup vs baseline: 22.9319x; 1.0907x over previous
import functools

import jax
import jax.numpy as jnp
from jax import lax
from jax.experimental import pallas as pl
from jax.experimental.pallas import tpu as pltpu
from jax.experimental.pallas import tpu_sc as plsc

N = 10000
D = 128
E = 320000
L = 3
C = 10
NB3 = 3

NC = 2
NS = 16
NW = NC * NS

NP = 10240
CH = 128
EPW = 10240
NCH = EPW // CH
EP = NW * EPW
RPT = NP // NS
ZCOPIES = RPT // CH
NCHH = NCH // 2


def _deg_body(dst_hbm, degp_hbm, idx_all, ones_v, zed_v, deg_sh, sem):
    c = lax.axis_index("c")
    s = lax.axis_index("s")
    wid = s * NC + c

    for j in range(CH // 16):
        ones_v[pl.ds(j * 16, 16)] = jnp.ones((16,), jnp.float32)

    def _zfill(i, carry):
        zed_v[pl.ds(i * 16, 16)] = jnp.zeros((16,), jnp.float32)
        return carry

    lax.fori_loop(0, RPT // 16, _zfill, 0)

    for b in range(NB3):
        pltpu.sync_copy(dst_hbm.at[b * NW + wid], idx_all)
        pltpu.sync_copy(zed_v, deg_sh.at[pl.ds(s * RPT, RPT)])
        plsc.subcore_barrier()

        def _fire(k, carry):
            pltpu.async_copy(ones_v, deg_sh.at[idx_all.at[k]], sem, add=True)
            return carry

        lax.fori_loop(0, NCH, _fire, 0)

        def _drain(k, carry):
            pltpu.make_async_copy(
                ones_v, deg_sh.at[idx_all.at[k]], sem).wait()
            return carry

        lax.fori_loop(0, NCH, _drain, 0)
        plsc.subcore_barrier()
        pltpu.sync_copy(deg_sh.at[pl.ds(s * RPT, RPT)],
                        degp_hbm.at[pl.ds((b * NC + c) * NP + s * RPT, RPT)])
        plsc.subcore_barrier()


def _acc_body(u_hbm, src_hbm, dst_hbm, accp_hbm,
              src_half, dst_half, rows_a, rows_b, acc_sh,
              sem_a, sem_b, sem_w):
    c = lax.axis_index("c")
    s = lax.axis_index("s")
    wid = s * NC + c

    def _zfill(i, carry):
        for j in range(D // 16):
            rows_a[i, pl.ds(j * 16, 16)] = jnp.zeros((16,), jnp.float32)
        return carry

    lax.fori_loop(0, CH, _zfill, 0)
    for r in range(ZCOPIES):
        pltpu.sync_copy(rows_a, acc_sh.at[pl.ds(s * RPT + r * CH, CH), :])
    plsc.subcore_barrier()

    for h in range(2):
        pltpu.sync_copy(src_hbm.at[wid, pl.ds(h * NCHH, NCHH), :], src_half)
        pltpu.sync_copy(dst_hbm.at[wid, pl.ds(h * NCHH, NCHH), :], dst_half)

        pltpu.async_copy(u_hbm.at[src_half.at[0]], rows_a, sem_a)
        pltpu.async_copy(u_hbm.at[src_half.at[1]], rows_b, sem_b)

        def _pipe(g, carry):
            k = 2 * g
            pltpu.make_async_copy(u_hbm.at[src_half.at[k]],
                                  rows_a, sem_a).wait()
            pltpu.sync_copy(rows_a, acc_sh.at[dst_half.at[k]], add=True)

            @pl.when(k + 2 < NCHH)
            def _():
                pltpu.async_copy(u_hbm.at[src_half.at[k + 2]], rows_a, sem_a)

            pltpu.make_async_copy(u_hbm.at[src_half.at[k + 1]],
                                  rows_b, sem_b).wait()
            pltpu.sync_copy(rows_b, acc_sh.at[dst_half.at[k + 1]], add=True)

            @pl.when(k + 3 < NCHH)
            def _():
                pltpu.async_copy(u_hbm.at[src_half.at[k + 3]], rows_b, sem_b)

            return carry

        lax.fori_loop(0, NCHH // 2, _pipe, 0)

    plsc.subcore_barrier()
    pltpu.sync_copy(acc_sh.at[pl.ds(s * RPT, RPT), :],
                    accp_hbm.at[pl.ds(c * NP + s * RPT, RPT), :])


@functools.cache
def _build_sc_kernels():
    mesh = plsc.VectorSubcoreMesh(
        core_axis_name="c", subcore_axis_name="s",
        num_cores=NC, num_subcores=NS)
    deg_k = pl.kernel(
        _deg_body,
        out_type=jax.ShapeDtypeStruct((NB3 * NC * NP,), jnp.float32),
        mesh=mesh,
        scratch_types=[
            pltpu.VMEM((NCH, CH), jnp.int32),
            pltpu.VMEM((CH,), jnp.float32),
            pltpu.VMEM((RPT,), jnp.float32),
            pltpu.VMEM_SHARED((NP,), jnp.float32),
            pltpu.SemaphoreType.DMA,
        ],
    )
    acc_k = pl.kernel(
        _acc_body,
        out_type=jax.ShapeDtypeStruct((NC * NP, D), jnp.float32),
        mesh=mesh,
        scratch_types=[
            pltpu.VMEM((NCHH, CH), jnp.int32),
            pltpu.VMEM((NCHH, CH), jnp.int32),
            pltpu.VMEM((CH, D), jnp.float32),
            pltpu.VMEM((CH, D), jnp.float32),
            pltpu.VMEM_SHARED((NP, D), jnp.float32),
            pltpu.SemaphoreType.DMA,
            pltpu.SemaphoreType.DMA,
            pltpu.SemaphoreType.DMA,
        ],
    )
    return deg_k, acc_k


def _sc_degrees(dst_f):
    degp = _build_sc_kernels()[0](dst_f.reshape(NB3 * NW, NCH, CH))
    return degp.reshape(NB3, NC, NP)


def _sc_accumulate(u_b, src_b, dst_b):
    accp = _build_sc_kernels()[1](u_b, src_b, dst_b)
    return accp.reshape(NC, NP, D)


RB = 1280
NBLK = NP // RB
EPS = 1e-5


def _init_body(x_ref, w_ref, degp_ref, u_ref, dinv_ref):
    deg = degp_ref[0] + degp_ref[1] + 1.0
    dinv = lax.rsqrt(deg)
    h = jnp.dot(x_ref[...], w_ref[...], preferred_element_type=jnp.float32)
    u_ref[...] = h * dinv
    dinv_ref[...] = dinv


def _layer_update(x, u, acc0, acc1, dinv, bb, gg, be):
    a = (acc0 + acc1 + u) * dinv + bb
    mu = jnp.mean(a, axis=-1, keepdims=True)
    var = jnp.mean((a - mu) ** 2, axis=-1, keepdims=True)
    hn = (a - mu) * lax.rsqrt(var + EPS) * gg + be
    return x + jnp.maximum(hn, 0.0)


def _mid_body(x_ref, u_ref, accp_ref, dinv_ref, b_ref, g_ref, be_ref, wn_ref,
              xn_ref, un_ref):
    dinv = dinv_ref[...]
    xn = _layer_update(x_ref[...], u_ref[...], accp_ref[0], accp_ref[1],
                       dinv, b_ref[...], g_ref[...], be_ref[...])
    xn_ref[...] = xn
    un_ref[...] = jnp.dot(xn, wn_ref[...],
                          preferred_element_type=jnp.float32) * dinv


def _fin_body(x_ref, u_ref, accp_ref, dinv_ref, b_ref, g_ref, be_ref,
              cw_ref, lg_ref):
    xn = _layer_update(x_ref[...], u_ref[...], accp_ref[0], accp_ref[1],
                       dinv_ref[...], b_ref[...], g_ref[...], be_ref[...])
    lg_ref[...] = jnp.dot(xn, cw_ref[...], preferred_element_type=jnp.float32)


def _sum_body(l0_ref, l1_ref, l2_ref, cb_ref, out_ref):
    out_ref[...] = l0_ref[...] + l1_ref[...] + l2_ref[...] + cb_ref[...]


_f_init = pl.pallas_call(
    _init_body,
    grid=(NBLK,),
    in_specs=[
        pl.BlockSpec((RB, D), lambda i: (i, 0)),
        pl.BlockSpec((D, D), lambda i: (0, 0)),
        pl.BlockSpec((NC, RB, 1), lambda i: (0, i, 0)),
    ],
    out_specs=[
        pl.BlockSpec((RB, D), lambda i: (i, 0)),
        pl.BlockSpec((RB, 1), lambda i: (i, 0)),
    ],
    out_shape=[
        jax.ShapeDtypeStruct((NP, D), jnp.float32),
        jax.ShapeDtypeStruct((NP, 1), jnp.float32),
    ],
)

_f_mid = pl.pallas_call(
    _mid_body,
    grid=(NBLK,),
    in_specs=[
        pl.BlockSpec((RB, D), lambda i: (i, 0)),
        pl.BlockSpec((RB, D), lambda i: (i, 0)),
        pl.BlockSpec((NC, RB, D), lambda i: (0, i, 0)),
        pl.BlockSpec((RB, 1), lambda i: (i, 0)),
        pl.BlockSpec((1, D), lambda i: (0, 0)),
        pl.BlockSpec((1, D), lambda i: (0, 0)),
        pl.BlockSpec((1, D), lambda i: (0, 0)),
        pl.BlockSpec((D, D), lambda i: (0, 0)),
    ],
    out_specs=[
        pl.BlockSpec((RB, D), lambda i: (i, 0)),
        pl.BlockSpec((RB, D), lambda i: (i, 0)),
    ],
    out_shape=[
        jax.ShapeDtypeStruct((NP, D), jnp.float32),
        jax.ShapeDtypeStruct((NP, D), jnp.float32),
    ],
)

_f_fin = pl.pallas_call(
    _fin_body,
    grid=(NBLK,),
    in_specs=[
        pl.BlockSpec((RB, D), lambda i: (i, 0)),
        pl.BlockSpec((RB, D), lambda i: (i, 0)),
        pl.BlockSpec((NC, RB, D), lambda i: (0, i, 0)),
        pl.BlockSpec((RB, 1), lambda i: (i, 0)),
        pl.BlockSpec((1, D), lambda i: (0, 0)),
        pl.BlockSpec((1, D), lambda i: (0, 0)),
        pl.BlockSpec((1, D), lambda i: (0, 0)),
        pl.BlockSpec((D, C), lambda i: (0, 0)),
    ],
    out_specs=pl.BlockSpec((RB, C), lambda i: (i, 0)),
    out_shape=jax.ShapeDtypeStruct((NP, C), jnp.float32),
)

_f_sum = pl.pallas_call(
    _sum_body,
    grid=(NBLK,),
    in_specs=[
        pl.BlockSpec((RB, C), lambda i: (i, 0)),
        pl.BlockSpec((RB, C), lambda i: (i, 0)),
        pl.BlockSpec((RB, C), lambda i: (i, 0)),
        pl.BlockSpec((1, C), lambda i: (0, 0)),
    ],
    out_specs=pl.BlockSpec((RB, C), lambda i: (i, 0)),
    out_shape=jax.ShapeDtypeStruct((NP, C), jnp.float32),
)


def kernel(x_renormalized, edge_index_renormalized, x_vanilla,
           edge_index_vanilla, x_third, edge_index_third,
           W_ren, b_ren, g_ren, be_ren, W_van, b_van, g_van, be_van,
           W_thd, b_thd, g_thd, be_thd, clf_W, clf_b):
    xs = [jnp.pad(x, ((0, NP - N), (0, 0)))
          for x in (x_renormalized, x_vanilla, x_third)]
    eis = jnp.stack([edge_index_renormalized, edge_index_vanilla,
                     edge_index_third])

    pad_n = EP - E
    pad_idx = (jnp.arange(pad_n, dtype=jnp.int32) % 16) + N
    pad_blk = jnp.broadcast_to(pad_idx, (NB3, pad_n))
    src_p = jnp.concatenate([eis[:, 0, :], pad_blk], axis=1)
    dst_p = jnp.concatenate([eis[:, 1, :], pad_blk], axis=1)
    srcs = [src_p[b].reshape(NW, NCH, CH) for b in range(NB3)]
    dsts = [dst_p[b].reshape(NW, NCH, CH) for b in range(NB3)]

    Ws = [W_ren, W_van, W_thd]
    bs = [b_ren, b_van, b_thd]
    gs = [g_ren, g_van, g_thd]
    bes = [be_ren, be_van, be_thd]
    cw = clf_W.reshape(NB3, D, C)
    cb = clf_b.reshape(1, C)

    degp = _sc_degrees(dst_p)

    us, dinvs = [], []
    for b in range(NB3):
        u_b, dinv_b = _f_init(xs[b], Ws[b][0],
                              degp[b].reshape(NC, NP, 1))
        us.append(u_b)
        dinvs.append(dinv_b)

    logits = [None] * NB3
    for l in range(L):
        accps = [_sc_accumulate(us[b], srcs[b], dsts[b]) for b in range(NB3)]
        for b in range(NB3):
            bl = bs[b][l].reshape(1, D)
            gl = gs[b][l].reshape(1, D)
            bel = bes[b][l].reshape(1, D)
            if l < L - 1:
                xs[b], us[b] = _f_mid(xs[b], us[b], accps[b], dinvs[b],
                                      bl, gl, bel, Ws[b][l + 1])
            else:
                logits[b] = _f_fin(xs[b], us[b], accps[b], dinvs[b],
                                   bl, gl, bel, cw[b])

    out = _f_sum(logits[0], logits[1], logits[2], cb)
    return out[:N]


# speedup vs baseline: 23.5749x; 1.0280x over previous
import functools

import jax
import jax.numpy as jnp
from jax import lax
from jax.experimental import pallas as pl
from jax.experimental.pallas import tpu as pltpu
from jax.experimental.pallas import tpu_sc as plsc

N = 10000
D = 128
E = 320000
L = 3
C = 10
NB3 = 3

NC = 2
NS = 16
NW = NC * NS

NP = 10240
CH = 128
EPW = 10240
NCH = EPW // CH
EP = NW * EPW
RPT = NP // NS
ZCOPIES = RPT // CH
NCHH = NCH // 2


def _deg_body(dst_hbm, degp_hbm, idx_all, ones_v, zed_v, deg_sh, sem):
    c = lax.axis_index("c")
    s = lax.axis_index("s")
    wid = s * NC + c

    for j in range(CH // 16):
        ones_v[pl.ds(j * 16, 16)] = jnp.ones((16,), jnp.float32)

    def _zfill(i, carry):
        zed_v[pl.ds(i * 16, 16)] = jnp.zeros((16,), jnp.float32)
        return carry

    lax.fori_loop(0, RPT // 16, _zfill, 0)

    pltpu.sync_copy(dst_hbm.at[wid], idx_all)
    pltpu.sync_copy(zed_v, deg_sh.at[pl.ds(s * RPT, RPT)])
    plsc.subcore_barrier()

    def _fire(k, carry):
        pltpu.async_copy(ones_v, deg_sh.at[idx_all.at[k]], sem, add=True)
        return carry

    lax.fori_loop(0, NCH, _fire, 0)

    def _drain(k, carry):
        pltpu.make_async_copy(ones_v, deg_sh.at[idx_all.at[k]], sem).wait()
        return carry

    lax.fori_loop(0, NCH, _drain, 0)
    plsc.subcore_barrier()
    pltpu.sync_copy(deg_sh.at[pl.ds(s * RPT, RPT)],
                    degp_hbm.at[pl.ds(c * NP + s * RPT, RPT)])


def _acc_body(u_hbm, src_hbm, dst_hbm, accp_hbm,
              src_half, dst_half, rows_a, rows_b, acc_sh,
              sem_a, sem_b, sem_i):
    c = lax.axis_index("c")
    s = lax.axis_index("s")
    wid = s * NC + c

    pltpu.async_copy(src_hbm.at[wid, pl.ds(0, NCHH), :], src_half, sem_i)
    pltpu.async_copy(dst_hbm.at[wid, pl.ds(0, NCHH), :], dst_half, sem_i)

    def _zfill(i, carry):
        for j in range(D // 16):
            rows_a[i, pl.ds(j * 16, 16)] = jnp.zeros((16,), jnp.float32)
        return carry

    lax.fori_loop(0, CH, _zfill, 0)
    for r in range(ZCOPIES):
        pltpu.sync_copy(rows_a, acc_sh.at[pl.ds(s * RPT + r * CH, CH), :])
    pltpu.make_async_copy(
        src_hbm.at[wid, pl.ds(0, NCHH), :], src_half, sem_i).wait()
    pltpu.make_async_copy(
        dst_hbm.at[wid, pl.ds(0, NCHH), :], dst_half, sem_i).wait()
    plsc.subcore_barrier()

    for h in range(2):
        if h > 0:
            pltpu.sync_copy(
                src_hbm.at[wid, pl.ds(h * NCHH, NCHH), :], src_half)
            pltpu.sync_copy(
                dst_hbm.at[wid, pl.ds(h * NCHH, NCHH), :], dst_half)

        pltpu.async_copy(u_hbm.at[src_half.at[0]], rows_a, sem_a)
        pltpu.async_copy(u_hbm.at[src_half.at[1]], rows_b, sem_b)

        def _pipe(g, carry):
            k = 2 * g
            pltpu.make_async_copy(u_hbm.at[src_half.at[k]],
                                  rows_a, sem_a).wait()
            pltpu.sync_copy(rows_a, acc_sh.at[dst_half.at[k]], add=True)

            @pl.when(k + 2 < NCHH)
            def _():
                pltpu.async_copy(u_hbm.at[src_half.at[k + 2]], rows_a, sem_a)

            pltpu.make_async_copy(u_hbm.at[src_half.at[k + 1]],
                                  rows_b, sem_b).wait()
            pltpu.sync_copy(rows_b, acc_sh.at[dst_half.at[k + 1]], add=True)

            @pl.when(k + 3 < NCHH)
            def _():
                pltpu.async_copy(u_hbm.at[src_half.at[k + 3]], rows_b, sem_b)

            return carry

        lax.fori_loop(0, NCHH // 2, _pipe, 0)

    plsc.subcore_barrier()
    pltpu.sync_copy(acc_sh.at[pl.ds(s * RPT, RPT), :],
                    accp_hbm.at[pl.ds(c * NP + s * RPT, RPT), :])


@functools.cache
def _build_sc_kernels():
    mesh = plsc.VectorSubcoreMesh(
        core_axis_name="c", subcore_axis_name="s",
        num_cores=NC, num_subcores=NS)
    deg_k = pl.kernel(
        _deg_body,
        out_type=jax.ShapeDtypeStruct((NC * NP,), jnp.float32),
        mesh=mesh,
        scratch_types=[
            pltpu.VMEM((NCH, CH), jnp.int32),
            pltpu.VMEM((CH,), jnp.float32),
            pltpu.VMEM((RPT,), jnp.float32),
            pltpu.VMEM_SHARED((NP,), jnp.float32),
            pltpu.SemaphoreType.DMA,
        ],
    )
    acc_k = pl.kernel(
        _acc_body,
        out_type=jax.ShapeDtypeStruct((NC * NP, D), jnp.float32),
        mesh=mesh,
        scratch_types=[
            pltpu.VMEM((NCHH, CH), jnp.int32),
            pltpu.VMEM((NCHH, CH), jnp.int32),
            pltpu.VMEM((CH, D), jnp.float32),
            pltpu.VMEM((CH, D), jnp.float32),
            pltpu.VMEM_SHARED((NP, D), jnp.float32),
            pltpu.SemaphoreType.DMA,
            pltpu.SemaphoreType.DMA,
            pltpu.SemaphoreType.DMA,
        ],
    )
    return deg_k, acc_k


def _sc_degrees(dst_b):
    degp = _build_sc_kernels()[0](dst_b)
    return degp.reshape(NC, NP, 1)


def _sc_accumulate(u_b, src_b, dst_b):
    accp = _build_sc_kernels()[1](u_b, src_b, dst_b)
    return accp.reshape(NC, NP, D)


RB = 1280
NBLK = NP // RB
EPS = 1e-5


def _init_body(x_ref, w_ref, degp_ref, u_ref, dinv_ref):
    deg = degp_ref[0] + degp_ref[1] + 1.0
    dinv = lax.rsqrt(deg)
    h = jnp.dot(x_ref[...], w_ref[...], preferred_element_type=jnp.float32)
    u_ref[...] = h * dinv
    dinv_ref[...] = dinv


def _layer_update(x, u, acc0, acc1, dinv, bb, gg, be):
    a = (acc0 + acc1 + u) * dinv + bb
    mu = jnp.mean(a, axis=-1, keepdims=True)
    var = jnp.mean((a - mu) ** 2, axis=-1, keepdims=True)
    hn = (a - mu) * lax.rsqrt(var + EPS) * gg + be
    return x + jnp.maximum(hn, 0.0)


def _mid_body(x_ref, u_ref, accp_ref, dinv_ref, b_ref, g_ref, be_ref, wn_ref,
              xn_ref, un_ref):
    dinv = dinv_ref[...]
    xn = _layer_update(x_ref[...], u_ref[...], accp_ref[0], accp_ref[1],
                       dinv, b_ref[...], g_ref[...], be_ref[...])
    xn_ref[...] = xn
    un_ref[...] = jnp.dot(xn, wn_ref[...],
                          preferred_element_type=jnp.float32) * dinv


def _fin_body(x_ref, u_ref, accp_ref, dinv_ref, b_ref, g_ref, be_ref,
              cw_ref, lg_ref):
    xn = _layer_update(x_ref[...], u_ref[...], accp_ref[0], accp_ref[1],
                       dinv_ref[...], b_ref[...], g_ref[...], be_ref[...])
    lg_ref[...] = jnp.dot(xn, cw_ref[...], preferred_element_type=jnp.float32)


def _fin2_body(x_ref, u_ref, accp_ref, dinv_ref, b_ref, g_ref, be_ref,
               cw_ref, l0_ref, l1_ref, cb_ref, out_ref):
    xn = _layer_update(x_ref[...], u_ref[...], accp_ref[0], accp_ref[1],
                       dinv_ref[...], b_ref[...], g_ref[...], be_ref[...])
    out_ref[...] = (l0_ref[...] + l1_ref[...] + cb_ref[...]
                    + jnp.dot(xn, cw_ref[...],
                              preferred_element_type=jnp.float32))


_f_init = pl.pallas_call(
    _init_body,
    grid=(NBLK,),
    in_specs=[
        pl.BlockSpec((RB, D), lambda i: (i, 0)),
        pl.BlockSpec((D, D), lambda i: (0, 0)),
        pl.BlockSpec((NC, RB, 1), lambda i: (0, i, 0)),
    ],
    out_specs=[
        pl.BlockSpec((RB, D), lambda i: (i, 0)),
        pl.BlockSpec((RB, 1), lambda i: (i, 0)),
    ],
    out_shape=[
        jax.ShapeDtypeStruct((NP, D), jnp.float32),
        jax.ShapeDtypeStruct((NP, 1), jnp.float32),
    ],
)

_f_mid = pl.pallas_call(
    _mid_body,
    grid=(NBLK,),
    in_specs=[
        pl.BlockSpec((RB, D), lambda i: (i, 0)),
        pl.BlockSpec((RB, D), lambda i: (i, 0)),
        pl.BlockSpec((NC, RB, D), lambda i: (0, i, 0)),
        pl.BlockSpec((RB, 1), lambda i: (i, 0)),
        pl.BlockSpec((1, D), lambda i: (0, 0)),
        pl.BlockSpec((1, D), lambda i: (0, 0)),
        pl.BlockSpec((1, D), lambda i: (0, 0)),
        pl.BlockSpec((D, D), lambda i: (0, 0)),
    ],
    out_specs=[
        pl.BlockSpec((RB, D), lambda i: (i, 0)),
        pl.BlockSpec((RB, D), lambda i: (i, 0)),
    ],
    out_shape=[
        jax.ShapeDtypeStruct((NP, D), jnp.float32),
        jax.ShapeDtypeStruct((NP, D), jnp.float32),
    ],
)

_f_fin = pl.pallas_call(
    _fin_body,
    grid=(NBLK,),
    in_specs=[
        pl.BlockSpec((RB, D), lambda i: (i, 0)),
        pl.BlockSpec((RB, D), lambda i: (i, 0)),
        pl.BlockSpec((NC, RB, D), lambda i: (0, i, 0)),
        pl.BlockSpec((RB, 1), lambda i: (i, 0)),
        pl.BlockSpec((1, D), lambda i: (0, 0)),
        pl.BlockSpec((1, D), lambda i: (0, 0)),
        pl.BlockSpec((1, D), lambda i: (0, 0)),
        pl.BlockSpec((D, C), lambda i: (0, 0)),
    ],
    out_specs=pl.BlockSpec((RB, C), lambda i: (i, 0)),
    out_shape=jax.ShapeDtypeStruct((NP, C), jnp.float32),
)

_f_fin2 = pl.pallas_call(
    _fin2_body,
    grid=(NBLK,),
    in_specs=[
        pl.BlockSpec((RB, D), lambda i: (i, 0)),
        pl.BlockSpec((RB, D), lambda i: (i, 0)),
        pl.BlockSpec((NC, RB, D), lambda i: (0, i, 0)),
        pl.BlockSpec((RB, 1), lambda i: (i, 0)),
        pl.BlockSpec((1, D), lambda i: (0, 0)),
        pl.BlockSpec((1, D), lambda i: (0, 0)),
        pl.BlockSpec((1, D), lambda i: (0, 0)),
        pl.BlockSpec((D, C), lambda i: (0, 0)),
        pl.BlockSpec((RB, C), lambda i: (i, 0)),
        pl.BlockSpec((RB, C), lambda i: (i, 0)),
        pl.BlockSpec((1, C), lambda i: (0, 0)),
    ],
    out_specs=pl.BlockSpec((RB, C), lambda i: (i, 0)),
    out_shape=jax.ShapeDtypeStruct((NP, C), jnp.float32),
)


def kernel(x_renormalized, edge_index_renormalized, x_vanilla,
           edge_index_vanilla, x_third, edge_index_third,
           W_ren, b_ren, g_ren, be_ren, W_van, b_van, g_van, be_van,
           W_thd, b_thd, g_thd, be_thd, clf_W, clf_b):
    xs = [jnp.pad(x, ((0, NP - N), (0, 0)))
          for x in (x_renormalized, x_vanilla, x_third)]
    eis = jnp.stack([edge_index_renormalized, edge_index_vanilla,
                     edge_index_third])

    pad_n = EP - E
    pad_idx = (jnp.arange(pad_n, dtype=jnp.int32) % 16) + N
    pad_blk = jnp.broadcast_to(pad_idx, (NB3, pad_n))
    src_p = jnp.concatenate([eis[:, 0, :], pad_blk], axis=1)
    dst_p = jnp.concatenate([eis[:, 1, :], pad_blk], axis=1)
    srcs = [src_p[b].reshape(NW, NCH, CH) for b in range(NB3)]
    dsts = [dst_p[b].reshape(NW, NCH, CH) for b in range(NB3)]

    Ws = [W_ren, W_van, W_thd]
    bs = [b_ren, b_van, b_thd]
    gs = [g_ren, g_van, g_thd]
    bes = [be_ren, be_van, be_thd]
    cw = clf_W.reshape(NB3, D, C)
    cb = clf_b.reshape(1, C)

    us, dinvs = [], []
    for b in range(NB3):
        degp_b = _sc_degrees(dsts[b])
        u_b, dinv_b = _f_init(xs[b], Ws[b][0], degp_b)
        us.append(u_b)
        dinvs.append(dinv_b)

    logits = [None] * NB3
    for l in range(L):
        accps = [_sc_accumulate(us[b], srcs[b], dsts[b]) for b in range(NB3)]
        for b in range(NB3):
            bl = bs[b][l].reshape(1, D)
            gl = gs[b][l].reshape(1, D)
            bel = bes[b][l].reshape(1, D)
            if l < L - 1:
                xs[b], us[b] = _f_mid(xs[b], us[b], accps[b], dinvs[b],
                                      bl, gl, bel, Ws[b][l + 1])
            elif b < NB3 - 1:
                logits[b] = _f_fin(xs[b], us[b], accps[b], dinvs[b],
                                   bl, gl, bel, cw[b])
            else:
                logits[b] = _f_fin2(xs[b], us[b], accps[b], dinvs[b],
                                    bl, gl, bel, cw[b],
                                    logits[0], logits[1], cb)

    return logits[NB3 - 1][:N]
